# Initial kernel scaffold; baseline (speedup 1.0000x reference)
import jax
import jax.numpy as jnp
from jax.experimental import pallas as pl


def kernel(x):
    raise NotImplementedError("write your pallas kernel here")


# speedup vs baseline: 44.4766x; 44.4766x over previous
import functools

import jax
import jax.numpy as jnp
from jax import lax
from jax.experimental import pallas as pl
from jax.experimental.pallas import tpu as pltpu
from jax.experimental.pallas import tpu_sc as plsc

jax.config.update("jax_enable_x64", True)

_KNN_COL_TILE = 256

def _i0():
    return jnp.int32(0)


def _knn_kernel(x_ref, xt_ref, p1_ref, p2_ref, *, k):
    jt = pl.program_id(1)
    j_tile = p1_ref.shape[2]
    base = jt.astype(jnp.int32) * jnp.int32(j_tile)
    xb = x_ref[0, :, pl.ds(base, j_tile)]
    xall = x_ref[0]
    xtb = xt_ref[0]

    d0 = xtb[:, 0:1] - xb[0:1, :]
    dist = d0 * d0
    d1 = xtb[:, 1:2] - xb[1:2, :]
    dist = dist + d1 * d1
    d2 = xtb[:, 2:3] - xb[2:3, :]
    dist = dist + d2 * d2

    kk = jnp.int32(k)
    iota_i = jax.lax.broadcasted_iota(jnp.int32, (k, j_tile), 0)
    iota_j = jax.lax.broadcasted_iota(jnp.int32, (k, j_tile), 1) + base
    inf = jnp.float32(jnp.inf)
    dist = jnp.where(iota_i == iota_j, inf, dist)

    m1 = jnp.min(dist, axis=0, keepdims=True)
    i1 = jnp.min(jnp.where(dist == m1, iota_i, kk), axis=0, keepdims=True)
    dist2 = jnp.where(iota_i == i1, inf, dist)
    m2 = jnp.min(dist2, axis=0, keepdims=True)
    i2 = jnp.min(jnp.where(dist2 == m2, iota_i, kk), axis=0, keepdims=True)

    oh1 = (iota_i == i1).astype(jnp.float32)
    oh2 = (iota_i == i2).astype(jnp.float32)
    p1_ref[0] = jax.lax.dot(xall, oh1, precision=jax.lax.Precision.HIGHEST,
                            preferred_element_type=jnp.float32)
    p2_ref[0] = jax.lax.dot(xall, oh2, precision=jax.lax.Precision.HIGHEST,
                            preferred_element_type=jnp.float32)


def _gather_kernel(x_ref, maskc_ref, out_ref, *, k):
    m = maskc_ref[0]
    iota_r = jax.lax.broadcasted_iota(jnp.int32, (k, k), 0)
    iota_c = jax.lax.broadcasted_iota(jnp.int32, (k, k), 1)
    tri = (iota_c <= iota_r).astype(jnp.float32)
    c = jax.lax.dot(tri, m, precision=jax.lax.Precision.HIGHEST,
                    preferred_element_type=jnp.float32)
    n = c[k - 1, 0]
    rank = c - jnp.float32(1.0)

    jrow = jax.lax.broadcasted_iota(jnp.int32, (1, k), 1).astype(jnp.float32)
    q = jnp.floor(jrow / n)
    jmod = jrow - q * n
    jmod = jnp.where(jmod < jnp.float32(0.0), jmod + n, jmod)
    jmod = jnp.where(jmod < jnp.float32(0.0), jmod + n, jmod)
    jmod = jnp.where(jmod >= n, jmod - n, jmod)
    jmod = jnp.where(jmod >= n, jmod - n, jmod)

    sel = ((m > jnp.float32(0.5)) & (rank == jmod)).astype(jnp.float32)
    out = jax.lax.dot(x_ref[0], sel, precision=jax.lax.Precision.HIGHEST,
                      preferred_element_type=jnp.float32)
    out_ref[0] = out


def _knn_neighbours(x, xt, b, k):
    j = _KNN_COL_TILE
    grid = (b, k // j)
    out_shape = jax.ShapeDtypeStruct((b, 3, k), jnp.float32)
    return pl.pallas_call(
        functools.partial(_knn_kernel, k=k),
        grid=grid,
        in_specs=[
            pl.BlockSpec((1, 3, k), lambda bb, jt: (bb, _i0(), _i0())),
            pl.BlockSpec((1, k, 3), lambda bb, jt: (bb, _i0(), _i0())),
        ],
        out_specs=[
            pl.BlockSpec((1, 3, j), lambda bb, jt: (bb, _i0(), jt)),
            pl.BlockSpec((1, 3, j), lambda bb, jt: (bb, _i0(), jt)),
        ],
        out_shape=[out_shape, out_shape],
    )(x, xt)


def _modulo_gather(x, maskc, b, k):
    return pl.pallas_call(
        functools.partial(_gather_kernel, k=k),
        grid=(b,),
        in_specs=[
            pl.BlockSpec((1, 3, k), lambda bb: (bb, _i0(), _i0())),
            pl.BlockSpec((1, k, 1), lambda bb: (bb, _i0(), _i0())),
        ],
        out_specs=pl.BlockSpec((1, 3, k), lambda bb: (bb, _i0(), _i0())),
        out_shape=jax.ShapeDtypeStruct((b, 3, k), jnp.float32),
    )(x, maskc)


def _sc_gather_body(x_hbm, mask_hbm, out_hbm, mask_v, idx_v,
                    x0_v, x1_v, x2_v, o0_v, o1_v, o2_v, *, b, k, nc):
    wid = lax.axis_index("s") * nc + lax.axis_index("c")
    nvec = k // 16

    @pl.when(wid < b)
    def _():
        kk = jnp.int32(k)
        xbase = wid * jnp.int32(3 * k)
        pltpu.sync_copy(mask_hbm.at[pl.ds(wid * kk, k)], mask_v)
        pltpu.sync_copy(x_hbm.at[pl.ds(xbase, k)], x0_v)
        pltpu.sync_copy(x_hbm.at[pl.ds(xbase + kk, k)], x1_v)
        pltpu.sync_copy(x_hbm.at[pl.ds(xbase + jnp.int32(2 * k), k)], x2_v)

        def compact(t, off):
            mv = mask_v[pl.ds(t * 16, 16)]
            ranks = plsc.cumsum(mv) + (off - jnp.int32(1))
            ivec = lax.iota(jnp.int32, 16) + t * jnp.int32(16)
            plsc.store_scatter(idx_v, [ranks], ivec, mask=mv > jnp.int32(0))
            return off + jnp.sum(mv, dtype=jnp.int32)

        n = lax.fori_loop(jnp.int32(0), jnp.int32(nvec), compact, jnp.int32(0))

        def emit(t, carry):
            jv = lax.iota(jnp.int32, 16) + t * jnp.int32(16)
            jm = jv % n
            iv = plsc.load_gather(idx_v, [jm])
            sl = pl.ds(t * 16, 16)
            o0_v[sl] = plsc.load_gather(x0_v, [iv])
            o1_v[sl] = plsc.load_gather(x1_v, [iv])
            o2_v[sl] = plsc.load_gather(x2_v, [iv])
            return carry

        lax.fori_loop(jnp.int32(0), jnp.int32(nvec), emit, jnp.int32(0))
        pltpu.sync_copy(o0_v, out_hbm.at[pl.ds(xbase, k)])
        pltpu.sync_copy(o1_v, out_hbm.at[pl.ds(xbase + kk, k)])
        pltpu.sync_copy(o2_v, out_hbm.at[pl.ds(xbase + jnp.int32(2 * k), k)])


def _modulo_gather_sc(x, maski, b, k):
    info = plsc.get_sparse_core_info()
    nc = info.num_cores
    mesh = plsc.VectorSubcoreMesh(core_axis_name="c", subcore_axis_name="s")
    f32, i32 = jnp.float32, jnp.int32
    fn = functools.partial(
        pl.kernel,
        mesh=mesh,
        compiler_params=pltpu.CompilerParams(needs_layout_passes=False),
        out_type=jax.ShapeDtypeStruct((b * 3 * k,), f32),
        scratch_types=[
            pltpu.VMEM((k,), i32),
            pltpu.VMEM((k,), i32),
            pltpu.VMEM((k,), f32), pltpu.VMEM((k,), f32), pltpu.VMEM((k,), f32),
            pltpu.VMEM((k,), f32), pltpu.VMEM((k,), f32), pltpu.VMEM((k,), f32),
        ],
    )(functools.partial(_sc_gather_body, b=b, k=k, nc=nc))
    out_flat = fn(x.reshape(b * 3 * k), maski.reshape(b * k))
    return out_flat.reshape(b, 3, k)


def kernel(x):
    b, _, k = x.shape
    xt = jnp.transpose(x, (0, 2, 1))

    p1, p2 = _knn_neighbours(x, xt, b, k)

    x64 = x.astype(jnp.float64)
    p164 = p1.astype(jnp.float64)
    p264 = p2.astype(jnp.float64)
    xx = jnp.sum(x64 * x64, axis=1)
    d1 = (jnp.sum(p164 * p164, axis=1)
          + (-2.0) * jnp.sum(x64 * p164, axis=1)) + xx
    d2 = (jnp.sum(p264 * p264, axis=1)
          + (-2.0) * jnp.sum(x64 * p264, axis=1)) + xx
    value = (d1 + d2) * 0.5
    mean = jnp.mean(value, axis=-1)
    std = jnp.std(value, axis=-1, ddof=1)
    mask = value <= (mean + 1.1 * std)[:, None]

    return _modulo_gather_sc(x, mask.astype(jnp.int32), b, k)


# speedup vs baseline: 47.4808x; 1.0675x over previous
import functools

import jax
import jax.numpy as jnp
from jax import lax
from jax.experimental import pallas as pl
from jax.experimental.pallas import tpu as pltpu
from jax.experimental.pallas import tpu_sc as plsc

jax.config.update("jax_enable_x64", True)

_KNN_COL_TILE = 512

def _i0():
    return jnp.int32(0)


def _knn_kernel(x_ref, xt_ref, p1_ref, p2_ref, *, k):
    jt = pl.program_id(1)
    j_tile = p1_ref.shape[2]
    base = jt.astype(jnp.int32) * jnp.int32(j_tile)
    xb = x_ref[0, :, pl.ds(base, j_tile)]
    xall = x_ref[0]
    xtb = xt_ref[0]

    inner = jax.lax.dot(xtb, xb, precision=jax.lax.Precision.HIGHEST,
                        preferred_element_type=jnp.float32)
    xx_col = jnp.sum(xtb * xtb, axis=1, keepdims=True)
    xx_row = jnp.sum(xb * xb, axis=0, keepdims=True)
    dist = (xx_col - jnp.float32(2.0) * inner) + xx_row

    kk = jnp.int32(k)
    iota_i = jax.lax.broadcasted_iota(jnp.int32, (k, j_tile), 0)
    iota_j = jax.lax.broadcasted_iota(jnp.int32, (k, j_tile), 1) + base
    inf = jnp.float32(jnp.inf)
    dist = jnp.where(iota_i == iota_j, inf, dist)

    m1 = jnp.min(dist, axis=0, keepdims=True)
    i1 = jnp.min(jnp.where(dist == m1, iota_i, kk), axis=0, keepdims=True)
    oh1 = iota_i == i1
    dist2 = jnp.where(oh1, inf, dist)
    m2 = jnp.min(dist2, axis=0, keepdims=True)
    i2 = jnp.min(jnp.where(dist2 == m2, iota_i, kk), axis=0, keepdims=True)
    oh2 = iota_i == i2

    p1_ref[0] = jax.lax.dot(xall, oh1.astype(jnp.float32),
                            precision=jax.lax.Precision.HIGHEST,
                            preferred_element_type=jnp.float32)
    p2_ref[0] = jax.lax.dot(xall, oh2.astype(jnp.float32),
                            precision=jax.lax.Precision.HIGHEST,
                            preferred_element_type=jnp.float32)


def _gather_kernel(x_ref, maskc_ref, out_ref, *, k):
    m = maskc_ref[0]
    iota_r = jax.lax.broadcasted_iota(jnp.int32, (k, k), 0)
    iota_c = jax.lax.broadcasted_iota(jnp.int32, (k, k), 1)
    tri = (iota_c <= iota_r).astype(jnp.float32)
    c = jax.lax.dot(tri, m, precision=jax.lax.Precision.HIGHEST,
                    preferred_element_type=jnp.float32)
    n = c[k - 1, 0]
    rank = c - jnp.float32(1.0)

    jrow = jax.lax.broadcasted_iota(jnp.int32, (1, k), 1).astype(jnp.float32)
    q = jnp.floor(jrow / n)
    jmod = jrow - q * n
    jmod = jnp.where(jmod < jnp.float32(0.0), jmod + n, jmod)
    jmod = jnp.where(jmod < jnp.float32(0.0), jmod + n, jmod)
    jmod = jnp.where(jmod >= n, jmod - n, jmod)
    jmod = jnp.where(jmod >= n, jmod - n, jmod)

    sel = ((m > jnp.float32(0.5)) & (rank == jmod)).astype(jnp.float32)
    out = jax.lax.dot(x_ref[0], sel, precision=jax.lax.Precision.HIGHEST,
                      preferred_element_type=jnp.float32)
    out_ref[0] = out


def _knn_neighbours(x, xt, b, k):
    j = _KNN_COL_TILE
    grid = (b, k // j)
    out_shape = jax.ShapeDtypeStruct((b, 3, k), jnp.float32)
    return pl.pallas_call(
        functools.partial(_knn_kernel, k=k),
        grid=grid,
        in_specs=[
            pl.BlockSpec((1, 3, k), lambda bb, jt: (bb, _i0(), _i0())),
            pl.BlockSpec((1, k, 3), lambda bb, jt: (bb, _i0(), _i0())),
        ],
        out_specs=[
            pl.BlockSpec((1, 3, j), lambda bb, jt: (bb, _i0(), jt)),
            pl.BlockSpec((1, 3, j), lambda bb, jt: (bb, _i0(), jt)),
        ],
        out_shape=[out_shape, out_shape],
    )(x, xt)


def _modulo_gather(x, maskc, b, k):
    return pl.pallas_call(
        functools.partial(_gather_kernel, k=k),
        grid=(b,),
        in_specs=[
            pl.BlockSpec((1, 3, k), lambda bb: (bb, _i0(), _i0())),
            pl.BlockSpec((1, k, 1), lambda bb: (bb, _i0(), _i0())),
        ],
        out_specs=pl.BlockSpec((1, 3, k), lambda bb: (bb, _i0(), _i0())),
        out_shape=jax.ShapeDtypeStruct((b, 3, k), jnp.float32),
    )(x, maskc)


def _sc_gather_body(x_hbm, mask_hbm, out_hbm, mask_v, idx_v,
                    x0_v, x1_v, x2_v, o0_v, o1_v, o2_v, *, b, k, nc):
    wid = lax.axis_index("s") * nc + lax.axis_index("c")
    nvec = k // 16

    @pl.when(wid < b)
    def _():
        kk = jnp.int32(k)
        xbase = wid * jnp.int32(3 * k)
        pltpu.sync_copy(mask_hbm.at[pl.ds(wid * kk, k)], mask_v)
        pltpu.sync_copy(x_hbm.at[pl.ds(xbase, k)], x0_v)
        pltpu.sync_copy(x_hbm.at[pl.ds(xbase + kk, k)], x1_v)
        pltpu.sync_copy(x_hbm.at[pl.ds(xbase + jnp.int32(2 * k), k)], x2_v)

        def compact(t, off):
            mv = mask_v[pl.ds(t * 16, 16)]
            ranks = plsc.cumsum(mv) + (off - jnp.int32(1))
            ivec = lax.iota(jnp.int32, 16) + t * jnp.int32(16)
            plsc.store_scatter(idx_v, [ranks], ivec, mask=mv > jnp.int32(0))
            return off + jnp.sum(mv, dtype=jnp.int32)

        n = lax.fori_loop(jnp.int32(0), jnp.int32(nvec), compact, jnp.int32(0))

        def emit(t, carry):
            jv = lax.iota(jnp.int32, 16) + t * jnp.int32(16)
            jm = jv % n
            iv = plsc.load_gather(idx_v, [jm])
            sl = pl.ds(t * 16, 16)
            o0_v[sl] = plsc.load_gather(x0_v, [iv])
            o1_v[sl] = plsc.load_gather(x1_v, [iv])
            o2_v[sl] = plsc.load_gather(x2_v, [iv])
            return carry

        lax.fori_loop(jnp.int32(0), jnp.int32(nvec), emit, jnp.int32(0))
        pltpu.sync_copy(o0_v, out_hbm.at[pl.ds(xbase, k)])
        pltpu.sync_copy(o1_v, out_hbm.at[pl.ds(xbase + kk, k)])
        pltpu.sync_copy(o2_v, out_hbm.at[pl.ds(xbase + jnp.int32(2 * k), k)])


def _modulo_gather_sc(x, maski, b, k):
    info = plsc.get_sparse_core_info()
    nc = info.num_cores
    mesh = plsc.VectorSubcoreMesh(core_axis_name="c", subcore_axis_name="s")
    f32, i32 = jnp.float32, jnp.int32
    fn = functools.partial(
        pl.kernel,
        mesh=mesh,
        compiler_params=pltpu.CompilerParams(needs_layout_passes=False),
        out_type=jax.ShapeDtypeStruct((b * 3 * k,), f32),
        scratch_types=[
            pltpu.VMEM((k,), i32),
            pltpu.VMEM((k,), i32),
            pltpu.VMEM((k,), f32), pltpu.VMEM((k,), f32), pltpu.VMEM((k,), f32),
            pltpu.VMEM((k,), f32), pltpu.VMEM((k,), f32), pltpu.VMEM((k,), f32),
        ],
    )(functools.partial(_sc_gather_body, b=b, k=k, nc=nc))
    out_flat = fn(x.reshape(b * 3 * k), maski.reshape(b * k))
    return out_flat.reshape(b, 3, k)


def kernel(x):
    b, _, k = x.shape
    xt = jnp.transpose(x, (0, 2, 1))

    p1, p2 = _knn_neighbours(x, xt, b, k)

    x64 = x.astype(jnp.float64)
    p164 = p1.astype(jnp.float64)
    p264 = p2.astype(jnp.float64)
    value = jnp.sum((p164 * p164 + p264 * p264) * 0.5
                    - x64 * (p164 + p264) + x64 * x64, axis=1)
    kf = jnp.float64(k)
    sv = jnp.sum(value, axis=-1)
    svv = jnp.sum(value * value, axis=-1)
    mean = sv / kf
    std = jnp.sqrt(jnp.maximum(svv - kf * mean * mean, 0.0) / (kf - 1.0))
    mask = value <= (mean + 1.1 * std)[:, None]

    return _modulo_gather_sc(x, mask.astype(jnp.int32), b, k)


# speedup vs baseline: 59.1374x; 1.2455x over previous
import functools

import jax
import jax.numpy as jnp
from jax import lax
from jax.experimental import pallas as pl
from jax.experimental.pallas import tpu as pltpu
from jax.experimental.pallas import tpu_sc as plsc

jax.config.update("jax_enable_x64", True)

_KNN_COL_TILE = 512

def _i0():
    return jnp.int32(0)


def _knn_kernel(x_ref, xt_ref, p1_ref, p2_ref, *, k):
    jt = pl.program_id(1)
    j_tile = p1_ref.shape[2]
    base = jt.astype(jnp.int32) * jnp.int32(j_tile)
    xb = x_ref[0, :, pl.ds(base, j_tile)]
    xall = x_ref[0]
    xtb = xt_ref[0]

    inner = jax.lax.dot(xtb, xb, precision=jax.lax.Precision.HIGHEST,
                        preferred_element_type=jnp.float32)
    xx_col = jnp.sum(xtb * xtb, axis=1, keepdims=True)
    xx_row = jnp.sum(xb * xb, axis=0, keepdims=True)

    iota_i = jax.lax.broadcasted_iota(jnp.int32, (k, j_tile), 0)
    iota_j = jax.lax.broadcasted_iota(jnp.int32, (k, j_tile), 1) + base
    inf = jnp.float32(jnp.inf)
    dist = jnp.where(iota_i == iota_j, inf,
                     (xx_col - jnp.float32(2.0) * inner) + xx_row)

    i1 = lax.argmin(dist, axis=0, index_dtype=jnp.int32)[None, :]
    oh1 = iota_i == i1
    dist2 = jnp.where(oh1, inf, dist)
    i2 = lax.argmin(dist2, axis=0, index_dtype=jnp.int32)[None, :]
    oh2 = iota_i == i2

    bf16, f32 = jnp.bfloat16, jnp.float32
    xh = xall.astype(bf16)
    xr = xall - xh.astype(f32)
    xm = xr.astype(bf16)
    xl = (xr - xm.astype(f32)).astype(bf16)

    def _sel(oh):
        oh16 = oh.astype(bf16)
        ph = jax.lax.dot(xh, oh16, preferred_element_type=f32)
        pm = jax.lax.dot(xm, oh16, preferred_element_type=f32)
        pll = jax.lax.dot(xl, oh16, preferred_element_type=f32)
        return (ph + pm) + pll

    p1_ref[0] = _sel(oh1)
    p2_ref[0] = _sel(oh2)


def _gather_kernel(x_ref, maskc_ref, out_ref, *, k):
    m = maskc_ref[0]
    iota_r = jax.lax.broadcasted_iota(jnp.int32, (k, k), 0)
    iota_c = jax.lax.broadcasted_iota(jnp.int32, (k, k), 1)
    tri = (iota_c <= iota_r).astype(jnp.float32)
    c = jax.lax.dot(tri, m, precision=jax.lax.Precision.HIGHEST,
                    preferred_element_type=jnp.float32)
    n = c[k - 1, 0]
    rank = c - jnp.float32(1.0)

    jrow = jax.lax.broadcasted_iota(jnp.int32, (1, k), 1).astype(jnp.float32)
    q = jnp.floor(jrow / n)
    jmod = jrow - q * n
    jmod = jnp.where(jmod < jnp.float32(0.0), jmod + n, jmod)
    jmod = jnp.where(jmod < jnp.float32(0.0), jmod + n, jmod)
    jmod = jnp.where(jmod >= n, jmod - n, jmod)
    jmod = jnp.where(jmod >= n, jmod - n, jmod)

    sel = ((m > jnp.float32(0.5)) & (rank == jmod)).astype(jnp.float32)
    out = jax.lax.dot(x_ref[0], sel, precision=jax.lax.Precision.HIGHEST,
                      preferred_element_type=jnp.float32)
    out_ref[0] = out


def _knn_neighbours(x, xt, b, k):
    j = _KNN_COL_TILE
    grid = (b, k // j)
    out_shape = jax.ShapeDtypeStruct((b, 3, k), jnp.float32)
    return pl.pallas_call(
        functools.partial(_knn_kernel, k=k),
        grid=grid,
        in_specs=[
            pl.BlockSpec((1, 3, k), lambda bb, jt: (bb, _i0(), _i0())),
            pl.BlockSpec((1, k, 3), lambda bb, jt: (bb, _i0(), _i0())),
        ],
        out_specs=[
            pl.BlockSpec((1, 3, j), lambda bb, jt: (bb, _i0(), jt)),
            pl.BlockSpec((1, 3, j), lambda bb, jt: (bb, _i0(), jt)),
        ],
        out_shape=[out_shape, out_shape],
    )(x, xt)


def _modulo_gather(x, maskc, b, k):
    return pl.pallas_call(
        functools.partial(_gather_kernel, k=k),
        grid=(b,),
        in_specs=[
            pl.BlockSpec((1, 3, k), lambda bb: (bb, _i0(), _i0())),
            pl.BlockSpec((1, k, 1), lambda bb: (bb, _i0(), _i0())),
        ],
        out_specs=pl.BlockSpec((1, 3, k), lambda bb: (bb, _i0(), _i0())),
        out_shape=jax.ShapeDtypeStruct((b, 3, k), jnp.float32),
    )(x, maskc)


def _sc_gather_body(x_hbm, mask_hbm, out_hbm, mask_v, idx_v,
                    x0_v, x1_v, x2_v, o0_v, o1_v, o2_v, *, b, k, nc):
    wid = lax.axis_index("s") * nc + lax.axis_index("c")
    nvec = k // 16

    @pl.when(wid < b)
    def _():
        kk = jnp.int32(k)
        xbase = wid * jnp.int32(3 * k)
        pltpu.sync_copy(mask_hbm.at[pl.ds(wid * kk, k)], mask_v)
        pltpu.sync_copy(x_hbm.at[pl.ds(xbase, k)], x0_v)
        pltpu.sync_copy(x_hbm.at[pl.ds(xbase + kk, k)], x1_v)
        pltpu.sync_copy(x_hbm.at[pl.ds(xbase + jnp.int32(2 * k), k)], x2_v)

        def compact(t, off):
            mv = mask_v[pl.ds(t * 16, 16)]
            ranks = plsc.cumsum(mv) + (off - jnp.int32(1))
            ivec = lax.iota(jnp.int32, 16) + t * jnp.int32(16)
            plsc.store_scatter(idx_v, [ranks], ivec, mask=mv > jnp.int32(0))
            return off + jnp.sum(mv, dtype=jnp.int32)

        n = lax.fori_loop(jnp.int32(0), jnp.int32(nvec), compact, jnp.int32(0))

        def emit(t, carry):
            jv = lax.iota(jnp.int32, 16) + t * jnp.int32(16)
            jm = jv % n
            iv = plsc.load_gather(idx_v, [jm])
            sl = pl.ds(t * 16, 16)
            o0_v[sl] = plsc.load_gather(x0_v, [iv])
            o1_v[sl] = plsc.load_gather(x1_v, [iv])
            o2_v[sl] = plsc.load_gather(x2_v, [iv])
            return carry

        lax.fori_loop(jnp.int32(0), jnp.int32(nvec), emit, jnp.int32(0))
        pltpu.sync_copy(o0_v, out_hbm.at[pl.ds(xbase, k)])
        pltpu.sync_copy(o1_v, out_hbm.at[pl.ds(xbase + kk, k)])
        pltpu.sync_copy(o2_v, out_hbm.at[pl.ds(xbase + jnp.int32(2 * k), k)])


def _modulo_gather_sc(x, maski, b, k):
    info = plsc.get_sparse_core_info()
    nc = info.num_cores
    mesh = plsc.VectorSubcoreMesh(core_axis_name="c", subcore_axis_name="s")
    f32, i32 = jnp.float32, jnp.int32
    fn = functools.partial(
        pl.kernel,
        mesh=mesh,
        compiler_params=pltpu.CompilerParams(needs_layout_passes=False),
        out_type=jax.ShapeDtypeStruct((b * 3 * k,), f32),
        scratch_types=[
            pltpu.VMEM((k,), i32),
            pltpu.VMEM((k,), i32),
            pltpu.VMEM((k,), f32), pltpu.VMEM((k,), f32), pltpu.VMEM((k,), f32),
            pltpu.VMEM((k,), f32), pltpu.VMEM((k,), f32), pltpu.VMEM((k,), f32),
        ],
    )(functools.partial(_sc_gather_body, b=b, k=k, nc=nc))
    out_flat = fn(x.reshape(b * 3 * k), maski.reshape(b * k))
    return out_flat.reshape(b, 3, k)


def kernel(x):
    b, _, k = x.shape
    xt = jnp.transpose(x, (0, 2, 1))

    p1, p2 = _knn_neighbours(x, xt, b, k)

    x64 = x.astype(jnp.float64)
    p164 = p1.astype(jnp.float64)
    p264 = p2.astype(jnp.float64)
    value = jnp.sum((p164 * p164 + p264 * p264) * 0.5
                    - x64 * (p164 + p264) + x64 * x64, axis=1)
    kf = jnp.float64(k)
    sv = jnp.sum(value, axis=-1)
    svv = jnp.sum(value * value, axis=-1)
    mean = sv / kf
    std = jnp.sqrt(jnp.maximum(svv - kf * mean * mean, 0.0) / (kf - 1.0))
    mask = value <= (mean + 1.1 * std)[:, None]

    return _modulo_gather_sc(x, mask.astype(jnp.int32), b, k)


# speedup vs baseline: 65.1972x; 1.1025x over previous
import functools

import jax
import jax.numpy as jnp
from jax import lax
from jax.experimental import pallas as pl
from jax.experimental.pallas import tpu as pltpu
from jax.experimental.pallas import tpu_sc as plsc

jax.config.update("jax_enable_x64", True)

_KNN_COL_TILE = 1024

def _i0():
    return jnp.int32(0)


def _knn_kernel(x_ref, xt_ref, p1_ref, p2_ref, *, k):
    jt = pl.program_id(1)
    j_tile = p1_ref.shape[2]
    base = jt.astype(jnp.int32) * jnp.int32(j_tile)
    xb = x_ref[0, :, pl.ds(base, j_tile)]
    xall = x_ref[0]
    xtb = xt_ref[0]

    inner = jax.lax.dot(xtb, xb, precision=jax.lax.Precision.HIGHEST,
                        preferred_element_type=jnp.float32)
    xx_col = jnp.sum(xtb * xtb, axis=1, keepdims=True)
    xx_row = jnp.sum(xb * xb, axis=0, keepdims=True)

    iota_i = jax.lax.broadcasted_iota(jnp.int32, (k, j_tile), 0)
    iota_j = jax.lax.broadcasted_iota(jnp.int32, (k, j_tile), 1) + base
    inf = jnp.float32(jnp.inf)
    dist = jnp.where(iota_i == iota_j, inf,
                     (xx_col - jnp.float32(2.0) * inner) + xx_row)

    i1 = lax.argmin(dist, axis=0, index_dtype=jnp.int32)[None, :]
    oh1 = iota_i == i1
    dist2 = jnp.where(oh1, inf, dist)
    i2 = lax.argmin(dist2, axis=0, index_dtype=jnp.int32)[None, :]
    oh2 = iota_i == i2

    bf16, f32 = jnp.bfloat16, jnp.float32
    xh = xall.astype(bf16)
    xr = xall - xh.astype(f32)
    xm = xr.astype(bf16)
    xl = (xr - xm.astype(f32)).astype(bf16)

    def _sel(oh):
        oh16 = oh.astype(bf16)
        ph = jax.lax.dot(xh, oh16, preferred_element_type=f32)
        pm = jax.lax.dot(xm, oh16, preferred_element_type=f32)
        pll = jax.lax.dot(xl, oh16, preferred_element_type=f32)
        return (ph + pm) + pll

    p1_ref[0] = _sel(oh1)
    p2_ref[0] = _sel(oh2)


def _gather_kernel(x_ref, maskc_ref, out_ref, *, k):
    m = maskc_ref[0]
    iota_r = jax.lax.broadcasted_iota(jnp.int32, (k, k), 0)
    iota_c = jax.lax.broadcasted_iota(jnp.int32, (k, k), 1)
    tri = (iota_c <= iota_r).astype(jnp.float32)
    c = jax.lax.dot(tri, m, precision=jax.lax.Precision.HIGHEST,
                    preferred_element_type=jnp.float32)
    n = c[k - 1, 0]
    rank = c - jnp.float32(1.0)

    jrow = jax.lax.broadcasted_iota(jnp.int32, (1, k), 1).astype(jnp.float32)
    q = jnp.floor(jrow / n)
    jmod = jrow - q * n
    jmod = jnp.where(jmod < jnp.float32(0.0), jmod + n, jmod)
    jmod = jnp.where(jmod < jnp.float32(0.0), jmod + n, jmod)
    jmod = jnp.where(jmod >= n, jmod - n, jmod)
    jmod = jnp.where(jmod >= n, jmod - n, jmod)

    sel = ((m > jnp.float32(0.5)) & (rank == jmod)).astype(jnp.float32)
    out = jax.lax.dot(x_ref[0], sel, precision=jax.lax.Precision.HIGHEST,
                      preferred_element_type=jnp.float32)
    out_ref[0] = out


def _knn_neighbours(x, xt, b, k):
    j = _KNN_COL_TILE
    grid = (b, k // j)
    out_shape = jax.ShapeDtypeStruct((b, 3, k), jnp.float32)
    return pl.pallas_call(
        functools.partial(_knn_kernel, k=k),
        grid=grid,
        in_specs=[
            pl.BlockSpec((1, 3, k), lambda bb, jt: (bb, _i0(), _i0())),
            pl.BlockSpec((1, k, 3), lambda bb, jt: (bb, _i0(), _i0())),
        ],
        out_specs=[
            pl.BlockSpec((1, 3, j), lambda bb, jt: (bb, _i0(), jt)),
            pl.BlockSpec((1, 3, j), lambda bb, jt: (bb, _i0(), jt)),
        ],
        out_shape=[out_shape, out_shape],
    )(x, xt)


def _modulo_gather(x, maskc, b, k):
    return pl.pallas_call(
        functools.partial(_gather_kernel, k=k),
        grid=(b,),
        in_specs=[
            pl.BlockSpec((1, 3, k), lambda bb: (bb, _i0(), _i0())),
            pl.BlockSpec((1, k, 1), lambda bb: (bb, _i0(), _i0())),
        ],
        out_specs=pl.BlockSpec((1, 3, k), lambda bb: (bb, _i0(), _i0())),
        out_shape=jax.ShapeDtypeStruct((b, 3, k), jnp.float32),
    )(x, maskc)


def _sc_gather_body(x_hbm, mask_hbm, out_hbm, mask_v, idx_v,
                    x0_v, x1_v, x2_v, o0_v, o1_v, o2_v, *, b, k, nc):
    wid = lax.axis_index("s") * nc + lax.axis_index("c")
    nvec = k // 16

    @pl.when(wid < b)
    def _():
        kk = jnp.int32(k)
        xbase = wid * jnp.int32(3 * k)
        pltpu.sync_copy(mask_hbm.at[pl.ds(wid * kk, k)], mask_v)
        pltpu.sync_copy(x_hbm.at[pl.ds(xbase, k)], x0_v)
        pltpu.sync_copy(x_hbm.at[pl.ds(xbase + kk, k)], x1_v)
        pltpu.sync_copy(x_hbm.at[pl.ds(xbase + jnp.int32(2 * k), k)], x2_v)

        def compact(t, off):
            mv = mask_v[pl.ds(t * 16, 16)]
            ranks = plsc.cumsum(mv) + (off - jnp.int32(1))
            ivec = lax.iota(jnp.int32, 16) + t * jnp.int32(16)
            plsc.store_scatter(idx_v, [ranks], ivec, mask=mv > jnp.int32(0))
            return off + jnp.sum(mv, dtype=jnp.int32)

        n = lax.fori_loop(jnp.int32(0), jnp.int32(nvec), compact, jnp.int32(0))

        def emit(t, carry):
            jv = lax.iota(jnp.int32, 16) + t * jnp.int32(16)
            jm = jv % n
            iv = plsc.load_gather(idx_v, [jm])
            sl = pl.ds(t * 16, 16)
            o0_v[sl] = plsc.load_gather(x0_v, [iv])
            o1_v[sl] = plsc.load_gather(x1_v, [iv])
            o2_v[sl] = plsc.load_gather(x2_v, [iv])
            return carry

        lax.fori_loop(jnp.int32(0), jnp.int32(nvec), emit, jnp.int32(0))
        pltpu.sync_copy(o0_v, out_hbm.at[pl.ds(xbase, k)])
        pltpu.sync_copy(o1_v, out_hbm.at[pl.ds(xbase + kk, k)])
        pltpu.sync_copy(o2_v, out_hbm.at[pl.ds(xbase + jnp.int32(2 * k), k)])


def _modulo_gather_sc(x, maski, b, k):
    info = plsc.get_sparse_core_info()
    nc = info.num_cores
    mesh = plsc.VectorSubcoreMesh(core_axis_name="c", subcore_axis_name="s")
    f32, i32 = jnp.float32, jnp.int32
    fn = functools.partial(
        pl.kernel,
        mesh=mesh,
        compiler_params=pltpu.CompilerParams(needs_layout_passes=False),
        out_type=jax.ShapeDtypeStruct((b * 3 * k,), f32),
        scratch_types=[
            pltpu.VMEM((k,), i32),
            pltpu.VMEM((k,), i32),
            pltpu.VMEM((k,), f32), pltpu.VMEM((k,), f32), pltpu.VMEM((k,), f32),
            pltpu.VMEM((k,), f32), pltpu.VMEM((k,), f32), pltpu.VMEM((k,), f32),
        ],
    )(functools.partial(_sc_gather_body, b=b, k=k, nc=nc))
    out_flat = fn(x.reshape(b * 3 * k), maski.reshape(b * k))
    return out_flat.reshape(b, 3, k)


def kernel(x):
    b, _, k = x.shape
    xt = jnp.transpose(x, (0, 2, 1))

    p1, p2 = _knn_neighbours(x, xt, b, k)

    x64 = x.astype(jnp.float64)
    p164 = p1.astype(jnp.float64)
    p264 = p2.astype(jnp.float64)
    value = jnp.sum((p164 * p164 + p264 * p264) * 0.5
                    - x64 * (p164 + p264) + x64 * x64, axis=1)
    kf = jnp.float64(k)
    sv = jnp.sum(value, axis=-1)
    svv = jnp.sum(value * value, axis=-1)
    mean = sv / kf
    std = jnp.sqrt(jnp.maximum(svv - kf * mean * mean, 0.0) / (kf - 1.0))
    mask = value <= (mean + 1.1 * std)[:, None]

    return _modulo_gather_sc(x, mask.astype(jnp.int32), b, k)


# speedup vs baseline: 66.6200x; 1.0218x over previous
import functools

import jax
import jax.numpy as jnp
from jax import lax
from jax.experimental import pallas as pl
from jax.experimental.pallas import tpu as pltpu
from jax.experimental.pallas import tpu_sc as plsc

jax.config.update("jax_enable_x64", True)

_KNN_COL_TILE = 1024

def _i0():
    return jnp.int32(0)


def _knn_kernel(x_ref, xt_ref, p1_ref, p2_ref, *, k):
    jt = pl.program_id(1)
    j_tile = p1_ref.shape[2]
    base = jt.astype(jnp.int32) * jnp.int32(j_tile)
    xb = x_ref[0, :, pl.ds(base, j_tile)]
    xall = x_ref[0]
    xtb = xt_ref[0]

    inner = jax.lax.dot(xtb, xb, precision=jax.lax.Precision.HIGHEST,
                        preferred_element_type=jnp.float32)
    xx_col = jnp.sum(xtb * xtb, axis=1, keepdims=True)
    xx_row = jnp.sum(xb * xb, axis=0, keepdims=True)

    iota_i = jax.lax.broadcasted_iota(jnp.int32, (k, j_tile), 0)
    iota_j = jax.lax.broadcasted_iota(jnp.int32, (k, j_tile), 1) + base
    inf = jnp.float32(jnp.inf)
    dist = jnp.where(iota_i == iota_j, inf,
                     (xx_col - jnp.float32(2.0) * inner) + xx_row)

    i1 = lax.argmin(dist, axis=0, index_dtype=jnp.int32)[None, :]
    oh1 = iota_i == i1
    dist2 = jnp.where(oh1, inf, dist)
    i2 = lax.argmin(dist2, axis=0, index_dtype=jnp.int32)[None, :]
    oh2 = iota_i == i2

    bf16, f32 = jnp.bfloat16, jnp.float32
    xh = xall.astype(bf16)
    xr = xall - xh.astype(f32)
    xm = xr.astype(bf16)
    xl = (xr - xm.astype(f32)).astype(bf16)

    def _sel(oh):
        oh16 = oh.astype(bf16)
        ph = jax.lax.dot(xh, oh16, preferred_element_type=f32)
        pm = jax.lax.dot(xm, oh16, preferred_element_type=f32)
        pll = jax.lax.dot(xl, oh16, preferred_element_type=f32)
        return (ph + pm) + pll

    p1_ref[0] = _sel(oh1)
    p2_ref[0] = _sel(oh2)


def _gather_kernel(x_ref, maskc_ref, out_ref, *, k):
    m = maskc_ref[0]
    iota_r = jax.lax.broadcasted_iota(jnp.int32, (k, k), 0)
    iota_c = jax.lax.broadcasted_iota(jnp.int32, (k, k), 1)
    tri = (iota_c <= iota_r).astype(jnp.float32)
    c = jax.lax.dot(tri, m, precision=jax.lax.Precision.HIGHEST,
                    preferred_element_type=jnp.float32)
    n = c[k - 1, 0]
    rank = c - jnp.float32(1.0)

    jrow = jax.lax.broadcasted_iota(jnp.int32, (1, k), 1).astype(jnp.float32)
    q = jnp.floor(jrow / n)
    jmod = jrow - q * n
    jmod = jnp.where(jmod < jnp.float32(0.0), jmod + n, jmod)
    jmod = jnp.where(jmod < jnp.float32(0.0), jmod + n, jmod)
    jmod = jnp.where(jmod >= n, jmod - n, jmod)
    jmod = jnp.where(jmod >= n, jmod - n, jmod)

    sel = ((m > jnp.float32(0.5)) & (rank == jmod)).astype(jnp.float32)
    out = jax.lax.dot(x_ref[0], sel, precision=jax.lax.Precision.HIGHEST,
                      preferred_element_type=jnp.float32)
    out_ref[0] = out


def _knn_neighbours(x, xt, b, k):
    j = _KNN_COL_TILE
    grid = (b, k // j)
    out_shape = jax.ShapeDtypeStruct((b, 3, k), jnp.float32)
    return pl.pallas_call(
        functools.partial(_knn_kernel, k=k),
        grid=grid,
        in_specs=[
            pl.BlockSpec((1, 3, k), lambda bb, jt: (bb, _i0(), _i0())),
            pl.BlockSpec((1, k, 3), lambda bb, jt: (bb, _i0(), _i0())),
        ],
        out_specs=[
            pl.BlockSpec((1, 3, j), lambda bb, jt: (bb, _i0(), jt)),
            pl.BlockSpec((1, 3, j), lambda bb, jt: (bb, _i0(), jt)),
        ],
        out_shape=[out_shape, out_shape],
    )(x, xt)


def _modulo_gather(x, maskc, b, k):
    return pl.pallas_call(
        functools.partial(_gather_kernel, k=k),
        grid=(b,),
        in_specs=[
            pl.BlockSpec((1, 3, k), lambda bb: (bb, _i0(), _i0())),
            pl.BlockSpec((1, k, 1), lambda bb: (bb, _i0(), _i0())),
        ],
        out_specs=pl.BlockSpec((1, 3, k), lambda bb: (bb, _i0(), _i0())),
        out_shape=jax.ShapeDtypeStruct((b, 3, k), jnp.float32),
    )(x, maskc)


def _sc_gather_body(x_hbm, mask_hbm, out_hbm, mask_v, idx_v,
                    x0_v, x1_v, x2_v, o0_v, o1_v, o2_v, sem, *, b, k, nc):
    wid = lax.axis_index("s") * nc + lax.axis_index("c")
    nvec = k // 16

    @pl.when(wid < b)
    def _():
        kk = jnp.int32(k)
        xbase = wid * jnp.int32(3 * k)
        h0 = pltpu.make_async_copy(x_hbm.at[pl.ds(xbase, k)], x0_v, sem)
        h1 = pltpu.make_async_copy(x_hbm.at[pl.ds(xbase + kk, k)], x1_v, sem)
        h2 = pltpu.make_async_copy(
            x_hbm.at[pl.ds(xbase + jnp.int32(2 * k), k)], x2_v, sem)
        h0.start()
        h1.start()
        h2.start()
        pltpu.sync_copy(mask_hbm.at[pl.ds(wid * kk, k)], mask_v)

        def compact(t, off):
            mv = mask_v[pl.ds(t * 16, 16)]
            ranks = plsc.cumsum(mv) + (off - jnp.int32(1))
            ivec = lax.iota(jnp.int32, 16) + t * jnp.int32(16)
            plsc.store_scatter(idx_v, [ranks], ivec, mask=mv > jnp.int32(0))
            return off + jnp.sum(mv, dtype=jnp.int32)

        n = lax.fori_loop(jnp.int32(0), jnp.int32(nvec), compact, jnp.int32(0))
        h0.wait()
        h1.wait()
        h2.wait()

        def emit(t, carry):
            jv = lax.iota(jnp.int32, 16) + t * jnp.int32(16)
            jm = jv % n
            iv = plsc.load_gather(idx_v, [jm])
            sl = pl.ds(t * 16, 16)
            o0_v[sl] = plsc.load_gather(x0_v, [iv])
            o1_v[sl] = plsc.load_gather(x1_v, [iv])
            o2_v[sl] = plsc.load_gather(x2_v, [iv])
            return carry

        lax.fori_loop(jnp.int32(0), jnp.int32(nvec), emit, jnp.int32(0))
        pltpu.sync_copy(o0_v, out_hbm.at[pl.ds(xbase, k)])
        pltpu.sync_copy(o1_v, out_hbm.at[pl.ds(xbase + kk, k)])
        pltpu.sync_copy(o2_v, out_hbm.at[pl.ds(xbase + jnp.int32(2 * k), k)])


def _modulo_gather_sc(x, maski, b, k):
    info = plsc.get_sparse_core_info()
    nc = info.num_cores
    mesh = plsc.VectorSubcoreMesh(core_axis_name="c", subcore_axis_name="s")
    f32, i32 = jnp.float32, jnp.int32
    fn = functools.partial(
        pl.kernel,
        mesh=mesh,
        compiler_params=pltpu.CompilerParams(needs_layout_passes=False),
        out_type=jax.ShapeDtypeStruct((b * 3 * k,), f32),
        scratch_types=[
            pltpu.VMEM((k,), i32),
            pltpu.VMEM((k,), i32),
            pltpu.VMEM((k,), f32), pltpu.VMEM((k,), f32), pltpu.VMEM((k,), f32),
            pltpu.VMEM((k,), f32), pltpu.VMEM((k,), f32), pltpu.VMEM((k,), f32),
            pltpu.SemaphoreType.DMA,
        ],
    )(functools.partial(_sc_gather_body, b=b, k=k, nc=nc))
    out_flat = fn(x.reshape(b * 3 * k), maski.reshape(b * k))
    return out_flat.reshape(b, 3, k)


def kernel(x):
    b, _, k = x.shape
    xt = jnp.transpose(x, (0, 2, 1))

    p1, p2 = _knn_neighbours(x, xt, b, k)

    x64 = x.astype(jnp.float64)
    p164 = p1.astype(jnp.float64)
    p264 = p2.astype(jnp.float64)
    value = jnp.sum((p164 * p164 + p264 * p264) * 0.5
                    - x64 * (p164 + p264) + x64 * x64, axis=1)
    kf = jnp.float64(k)
    sv = jnp.sum(value, axis=-1)
    svv = jnp.sum(value * value, axis=-1)
    mean = sv / kf
    std = jnp.sqrt(jnp.maximum(svv - kf * mean * mean, 0.0) / (kf - 1.0))
    mask = value <= (mean + 1.1 * std)[:, None]

    return _modulo_gather_sc(x, mask.astype(jnp.int32), b, k)


# speedup vs baseline: 76.0605x; 1.1417x over previous
import functools

import jax
import jax.numpy as jnp
from jax import lax
from jax.experimental import pallas as pl
from jax.experimental.pallas import tpu as pltpu
from jax.experimental.pallas import tpu_sc as plsc

jax.config.update("jax_enable_x64", True)

_KNN_COL_TILE = 1024

def _i0():
    return jnp.int32(0)


def _knn_kernel(x_ref, xt_ref, p1_ref, p2_ref, *, k):
    jt = pl.program_id(1)
    j_tile = p1_ref.shape[2]
    base = jt.astype(jnp.int32) * jnp.int32(j_tile)
    xb = x_ref[0, :, pl.ds(base, j_tile)]
    xall = x_ref[0]
    xtb = xt_ref[0]

    inner = jax.lax.dot(xtb, xb, precision=jax.lax.Precision.HIGHEST,
                        preferred_element_type=jnp.float32)
    xx_col = jnp.sum(xtb * xtb, axis=1, keepdims=True)

    iota_i = jax.lax.broadcasted_iota(jnp.int32, (k, j_tile), 0)
    iota_j = jax.lax.broadcasted_iota(jnp.int32, (k, j_tile), 1) + base
    inf = jnp.float32(jnp.inf)
    dist = jnp.where(iota_i == iota_j, inf,
                     xx_col - jnp.float32(2.0) * inner)

    i1 = lax.argmin(dist, axis=0, index_dtype=jnp.int32)[None, :]
    oh1 = iota_i == i1
    dist2 = jnp.where(oh1, inf, dist)
    i2 = lax.argmin(dist2, axis=0, index_dtype=jnp.int32)[None, :]
    oh2 = iota_i == i2

    bf16, f32 = jnp.bfloat16, jnp.float32
    xh = xall.astype(bf16)
    xr = xall - xh.astype(f32)
    xm = xr.astype(bf16)
    xl = (xr - xm.astype(f32)).astype(bf16)
    xcat = jnp.concatenate([xh, xm, xl], axis=0)

    def _sel(oh):
        pcat = jax.lax.dot(xcat, oh.astype(bf16), preferred_element_type=f32)
        return (pcat[0:3, :] + pcat[3:6, :]) + pcat[6:9, :]

    p1_ref[0] = _sel(oh1)
    p2_ref[0] = _sel(oh2)


def _gather_kernel(x_ref, maskc_ref, out_ref, *, k):
    m = maskc_ref[0]
    iota_r = jax.lax.broadcasted_iota(jnp.int32, (k, k), 0)
    iota_c = jax.lax.broadcasted_iota(jnp.int32, (k, k), 1)
    tri = (iota_c <= iota_r).astype(jnp.float32)
    c = jax.lax.dot(tri, m, precision=jax.lax.Precision.HIGHEST,
                    preferred_element_type=jnp.float32)
    n = c[k - 1, 0]
    rank = c - jnp.float32(1.0)

    jrow = jax.lax.broadcasted_iota(jnp.int32, (1, k), 1).astype(jnp.float32)
    q = jnp.floor(jrow / n)
    jmod = jrow - q * n
    jmod = jnp.where(jmod < jnp.float32(0.0), jmod + n, jmod)
    jmod = jnp.where(jmod < jnp.float32(0.0), jmod + n, jmod)
    jmod = jnp.where(jmod >= n, jmod - n, jmod)
    jmod = jnp.where(jmod >= n, jmod - n, jmod)

    sel = ((m > jnp.float32(0.5)) & (rank == jmod)).astype(jnp.float32)
    out = jax.lax.dot(x_ref[0], sel, precision=jax.lax.Precision.HIGHEST,
                      preferred_element_type=jnp.float32)
    out_ref[0] = out


def _knn_neighbours(x, xt, b, k):
    j = _KNN_COL_TILE
    grid = (b, k // j)
    out_shape = jax.ShapeDtypeStruct((b, 3, k), jnp.float32)
    return pl.pallas_call(
        functools.partial(_knn_kernel, k=k),
        grid=grid,
        in_specs=[
            pl.BlockSpec((1, 3, k), lambda bb, jt: (bb, _i0(), _i0())),
            pl.BlockSpec((1, k, 3), lambda bb, jt: (bb, _i0(), _i0())),
        ],
        out_specs=[
            pl.BlockSpec((1, 3, j), lambda bb, jt: (bb, _i0(), jt)),
            pl.BlockSpec((1, 3, j), lambda bb, jt: (bb, _i0(), jt)),
        ],
        out_shape=[out_shape, out_shape],
    )(x, xt)


def _modulo_gather(x, maskc, b, k):
    return pl.pallas_call(
        functools.partial(_gather_kernel, k=k),
        grid=(b,),
        in_specs=[
            pl.BlockSpec((1, 3, k), lambda bb: (bb, _i0(), _i0())),
            pl.BlockSpec((1, k, 1), lambda bb: (bb, _i0(), _i0())),
        ],
        out_specs=pl.BlockSpec((1, 3, k), lambda bb: (bb, _i0(), _i0())),
        out_shape=jax.ShapeDtypeStruct((b, 3, k), jnp.float32),
    )(x, maskc)


def _sc_gather_body(x_hbm, mask_hbm, out_hbm, mask_v, idx_v,
                    x0_v, x1_v, x2_v, o0_v, o1_v, o2_v, sem, *, b, k, nc):
    wid = lax.axis_index("s") * nc + lax.axis_index("c")
    nvec = k // 16

    @pl.when(wid < b)
    def _():
        kk = jnp.int32(k)
        xbase = wid * jnp.int32(3 * k)
        h0 = pltpu.make_async_copy(x_hbm.at[pl.ds(xbase, k)], x0_v, sem)
        h1 = pltpu.make_async_copy(x_hbm.at[pl.ds(xbase + kk, k)], x1_v, sem)
        h2 = pltpu.make_async_copy(
            x_hbm.at[pl.ds(xbase + jnp.int32(2 * k), k)], x2_v, sem)
        h0.start()
        h1.start()
        h2.start()
        pltpu.sync_copy(mask_hbm.at[pl.ds(wid * kk, k)], mask_v)

        def compact(t, off):
            mv = mask_v[pl.ds(t * 16, 16)]
            ranks = plsc.cumsum(mv) + (off - jnp.int32(1))
            ivec = lax.iota(jnp.int32, 16) + t * jnp.int32(16)
            plsc.store_scatter(idx_v, [ranks], ivec, mask=mv > jnp.int32(0))
            return off + jnp.sum(mv, dtype=jnp.int32)

        n = lax.fori_loop(jnp.int32(0), jnp.int32(nvec), compact, jnp.int32(0))
        h0.wait()
        h1.wait()
        h2.wait()

        def emit(t, carry):
            jv = lax.iota(jnp.int32, 16) + t * jnp.int32(16)
            jm = jv % n
            iv = plsc.load_gather(idx_v, [jm])
            sl = pl.ds(t * 16, 16)
            o0_v[sl] = plsc.load_gather(x0_v, [iv])
            o1_v[sl] = plsc.load_gather(x1_v, [iv])
            o2_v[sl] = plsc.load_gather(x2_v, [iv])
            return carry

        lax.fori_loop(jnp.int32(0), jnp.int32(nvec), emit, jnp.int32(0))
        pltpu.sync_copy(o0_v, out_hbm.at[pl.ds(xbase, k)])
        pltpu.sync_copy(o1_v, out_hbm.at[pl.ds(xbase + kk, k)])
        pltpu.sync_copy(o2_v, out_hbm.at[pl.ds(xbase + jnp.int32(2 * k), k)])


def _modulo_gather_sc(x, maski, b, k):
    info = plsc.get_sparse_core_info()
    nc = info.num_cores
    mesh = plsc.VectorSubcoreMesh(core_axis_name="c", subcore_axis_name="s")
    f32, i32 = jnp.float32, jnp.int32
    fn = functools.partial(
        pl.kernel,
        mesh=mesh,
        compiler_params=pltpu.CompilerParams(needs_layout_passes=False),
        out_type=jax.ShapeDtypeStruct((b * 3 * k,), f32),
        scratch_types=[
            pltpu.VMEM((k,), i32),
            pltpu.VMEM((k,), i32),
            pltpu.VMEM((k,), f32), pltpu.VMEM((k,), f32), pltpu.VMEM((k,), f32),
            pltpu.VMEM((k,), f32), pltpu.VMEM((k,), f32), pltpu.VMEM((k,), f32),
            pltpu.SemaphoreType.DMA,
        ],
    )(functools.partial(_sc_gather_body, b=b, k=k, nc=nc))
    out_flat = fn(x.reshape(b * 3 * k), maski.reshape(b * k))
    return out_flat.reshape(b, 3, k)


def kernel(x):
    b, _, k = x.shape
    xt = jnp.transpose(x, (0, 2, 1))

    p1, p2 = _knn_neighbours(x, xt, b, k)

    x64 = x.astype(jnp.float64)
    p164 = p1.astype(jnp.float64)
    p264 = p2.astype(jnp.float64)
    value = jnp.sum((p164 * p164 + p264 * p264) * 0.5
                    - x64 * (p164 + p264) + x64 * x64, axis=1)
    kf = jnp.float64(k)
    sv = jnp.sum(value, axis=-1)
    svv = jnp.sum(value * value, axis=-1)
    mean = sv / kf
    std = jnp.sqrt(jnp.maximum(svv - kf * mean * mean, 0.0) / (kf - 1.0))
    mask = value <= (mean + 1.1 * std)[:, None]

    return _modulo_gather_sc(x, mask.astype(jnp.int32), b, k)
